# baseline (speedup 1.0000x reference)
import functools

import jax
import jax.numpy as jnp
from jax.experimental import pallas as pl
from jax.experimental.pallas import tpu as pltpu


def _one_chamfer_tile(p4c, prc, pic, t4c, out, t2p_min, psum, pcnt,
                      j, npb, nt):
    p4 = p4c[0]
    t4 = t4c[0]
    pr = prc[0]
    pi = pic[0]
    pn = pr * pr + pi * pi
    valid = pn > 1e-12
    pnm = jnp.where(valid, pn, jnp.inf)
    tn = t4[0:1] * t4[0:1] + t4[1:2] * t4[1:2]
    cross = jax.lax.dot_general(
        p4, t4, (((1,), (0,)), ((), ())),
        precision=jax.lax.Precision.HIGHEST,
        preferred_element_type=jnp.float32)
    rowmin = jnp.min(cross + tn, axis=1, keepdims=True)
    blk_sum = jnp.sum(jnp.where(valid, rowmin + pn, 0.0))
    blk_cnt = jnp.sum(valid.astype(jnp.float32))
    colpart = jnp.min(cross + pnm, axis=0, keepdims=True)

    @pl.when(j == 0)
    def _init():
        t2p_min[...] = colpart
        psum[0, 0] = blk_sum
        pcnt[0, 0] = blk_cnt

    @pl.when(j != 0)
    def _acc():
        t2p_min[...] = jnp.minimum(t2p_min[...], colpart)
        psum[0, 0] = psum[0, 0] + blk_sum
        pcnt[0, 0] = pcnt[0, 0] + blk_cnt

    @pl.when(j == npb - 1)
    def _final():
        mean_p2t = psum[0, 0] / jnp.maximum(pcnt[0, 0], 1.0)
        mean_t2p = jnp.sum(t2p_min[...] + tn) / nt
        out[...] = jnp.reshape(mean_p2t + mean_t2p, (1, 1, 1))


def _both_kernel(pp4, pprc, ppic, pt4, zp4, zprc, zpic, zt4,
                 pole_out, zero_out,
                 p_t2p, p_sum, p_cnt, z_t2p, z_sum, z_cnt,
                 *, npb, ntp, ntz):
    j = pl.program_id(1)
    _one_chamfer_tile(pp4, pprc, ppic, pt4, pole_out, p_t2p, p_sum, p_cnt,
                      j, npb, float(ntp))
    _one_chamfer_tile(zp4, zprc, zpic, zt4, zero_out, z_t2p, z_sum, z_cnt,
                      j, npb, float(ntz))


def _pad8_pred(pred):
    b, n, _ = pred.shape
    return jnp.concatenate(
        [pred * jnp.float32(-2.0), jnp.zeros((b, n, 6), jnp.float32)], axis=2)


def _pad8_tgt(target):
    b, n, _ = target.shape
    t = jnp.transpose(target, (0, 2, 1))
    return jnp.concatenate([t, jnp.zeros((b, 6, n), jnp.float32)], axis=1)


def kernel(pred_poles, pred_zeros, target_poles_list, target_zeros_list):
    b, np_, _ = pred_poles.shape
    ntp = target_poles_list.shape[1]
    ntz = target_zeros_list.shape[1]
    pb = 1024
    npb = np_ // pb

    pp4 = _pad8_pred(pred_poles)
    zp4 = _pad8_pred(pred_zeros)
    pt4 = _pad8_tgt(target_poles_list)
    zt4 = _pad8_tgt(target_zeros_list)
    pprc, ppic = pred_poles[..., 0:1], pred_poles[..., 1:2]
    zprc, zpic = pred_zeros[..., 0:1], pred_zeros[..., 1:2]

    p4_spec = pl.BlockSpec((1, pb, 8), lambda i, j: (i, j, 0))
    col_spec = pl.BlockSpec((1, pb, 1), lambda i, j: (i, j, 0))
    pt4_spec = pl.BlockSpec((1, 8, ntp), lambda i, j: (i, 0, 0))
    zt4_spec = pl.BlockSpec((1, 8, ntz), lambda i, j: (i, 0, 0))
    out_spec = pl.BlockSpec((1, 1, 1), lambda i, j: (i, 0, 0))

    pole_losses, zero_losses = pl.pallas_call(
        functools.partial(_both_kernel, npb=npb, ntp=ntp, ntz=ntz),
        grid=(b, npb),
        in_specs=[p4_spec, col_spec, col_spec, pt4_spec,
                  p4_spec, col_spec, col_spec, zt4_spec],
        out_specs=[out_spec, out_spec],
        out_shape=[jax.ShapeDtypeStruct((b, 1, 1), jnp.float32),
                   jax.ShapeDtypeStruct((b, 1, 1), jnp.float32)],
        scratch_shapes=[
            pltpu.VMEM((1, ntp), jnp.float32),
            pltpu.SMEM((1, 1), jnp.float32),
            pltpu.SMEM((1, 1), jnp.float32),
            pltpu.VMEM((1, ntz), jnp.float32),
            pltpu.SMEM((1, 1), jnp.float32),
            pltpu.SMEM((1, 1), jnp.float32),
        ],
        compiler_params=pltpu.CompilerParams(
            dimension_semantics=("parallel", "arbitrary"),
        ),
    )(pp4, pprc, ppic, pt4, zp4, zprc, zpic, zt4)

    return jnp.mean(pole_losses) + 0.5 * jnp.mean(zero_losses)


# speedup vs baseline: 2.6727x; 2.6727x over previous
import functools

import jax
import jax.numpy as jnp
from jax.experimental import pallas as pl
from jax.experimental.pallas import tpu as pltpu


def _one_chamfer(prc, pic, trr, tir, cb, nt):
    pr = prc[0]
    pi = pic[0]
    pn = pr * pr + pi * pi
    valid = pn > 1e-12
    prm = jnp.where(valid, pr, jnp.inf)
    pim = jnp.where(valid, pi, jnp.inf)
    tr_all = trr[0]
    ti_all = tir[0]
    acc = None
    t2p_sum = jnp.float32(0.0)
    for c in range(nt // cb):
        tr = tr_all[:, c * cb:(c + 1) * cb]
        ti = ti_all[:, c * cb:(c + 1) * cb]
        dr = prm - tr
        di = pim - ti
        dist = dr * dr + di * di
        t2p_sum = t2p_sum + jnp.sum(jnp.min(dist, axis=0))
        acc = dist if acc is None else jnp.minimum(acc, dist)
    rowmin = jnp.min(acc, axis=1, keepdims=True)
    p2t_sum = jnp.sum(jnp.where(valid, rowmin, 0.0))
    cnt = jnp.sum(valid.astype(jnp.float32))
    return p2t_sum / jnp.maximum(cnt, 1.0) + t2p_sum / nt


def _both_kernel(pprc, ppic, ptrr, ptir, zprc, zpic, ztrr, ztir,
                 pole_out, zero_out, *, cb, ntp, ntz):
    pole_out[...] = jnp.reshape(
        _one_chamfer(pprc, ppic, ptrr, ptir, cb, ntp), (1, 1, 1))
    zero_out[...] = jnp.reshape(
        _one_chamfer(zprc, zpic, ztrr, ztir, cb, ntz), (1, 1, 1))


def _split_cols(pred):
    return pred[..., 0:1], pred[..., 1:2]


def _split_rows(target):
    tr = jnp.transpose(target[..., 0:1], (0, 2, 1))
    ti = jnp.transpose(target[..., 1:2], (0, 2, 1))
    return tr, ti


def kernel(pred_poles, pred_zeros, target_poles_list, target_zeros_list):
    b, np_, _ = pred_poles.shape
    ntp = target_poles_list.shape[1]
    ntz = target_zeros_list.shape[1]

    pprc, ppic = _split_cols(pred_poles)
    zprc, zpic = _split_cols(pred_zeros)
    ptrr, ptir = _split_rows(target_poles_list)
    ztrr, ztir = _split_rows(target_zeros_list)

    pred_spec = pl.BlockSpec((1, np_, 1), lambda i: (i, 0, 0))
    ptgt_spec = pl.BlockSpec((1, 1, ntp), lambda i: (i, 0, 0))
    ztgt_spec = pl.BlockSpec((1, 1, ntz), lambda i: (i, 0, 0))
    out_spec = pl.BlockSpec((1, 1, 1), lambda i: (i, 0, 0))

    pole_losses, zero_losses = pl.pallas_call(
        functools.partial(_both_kernel, cb=512, ntp=ntp, ntz=ntz),
        grid=(b,),
        in_specs=[pred_spec, pred_spec, ptgt_spec, ptgt_spec,
                  pred_spec, pred_spec, ztgt_spec, ztgt_spec],
        out_specs=[out_spec, out_spec],
        out_shape=[jax.ShapeDtypeStruct((b, 1, 1), jnp.float32),
                   jax.ShapeDtypeStruct((b, 1, 1), jnp.float32)],
        compiler_params=pltpu.CompilerParams(
            dimension_semantics=("parallel",),
        ),
    )(pprc, ppic, ptrr, ptir, zprc, zpic, ztrr, ztir)

    return jnp.mean(pole_losses) + 0.5 * jnp.mean(zero_losses)


# speedup vs baseline: 3.2061x; 1.1996x over previous
import functools

import jax
import jax.numpy as jnp
from jax.experimental import pallas as pl
from jax.experimental.pallas import tpu as pltpu


def _one_chamfer(prc, pic, trr, tir, cb, nt):
    pr = jnp.transpose(prc[0], (1, 0))
    pi = jnp.transpose(pic[0], (1, 0))
    pn = pr * pr + pi * pi
    valid = pn > 1e-12
    prm = jnp.where(valid, pr, jnp.inf)
    pim = jnp.where(valid, pi, jnp.inf)
    tr_all = trr[0]
    ti_all = tir[0]
    acc = None
    t2p_sum = jnp.float32(0.0)
    for c in range(nt // cb):
        tr = tr_all[:, c * cb:(c + 1) * cb]
        ti = ti_all[:, c * cb:(c + 1) * cb]
        dr = prm - tr
        di = pim - ti
        dist = dr * dr + di * di
        t2p_sum = t2p_sum + jnp.sum(jnp.min(dist, axis=0))
        acc = dist if acc is None else jnp.minimum(acc, dist)
    rowmin = jnp.min(acc, axis=1, keepdims=True)
    p2t_sum = jnp.sum(jnp.where(valid, rowmin, 0.0))
    cnt = jnp.sum(valid.astype(jnp.float32))
    return p2t_sum / jnp.maximum(cnt, 1.0) + t2p_sum / nt


def _both_kernel(pprc, ppic, ptrr, ptir, zprc, zpic, ztrr, ztir,
                 pole_out, zero_out, *, cb, ntp, ntz):
    pole_out[...] = jnp.reshape(
        _one_chamfer(pprc, ppic, ptrr, ptir, cb, ntp), (1, 1, 1))
    zero_out[...] = jnp.reshape(
        _one_chamfer(zprc, zpic, ztrr, ztir, cb, ntz), (1, 1, 1))


def _split_rows(target):
    tr = jnp.transpose(target[..., 0:1], (0, 2, 1))
    ti = jnp.transpose(target[..., 1:2], (0, 2, 1))
    return tr, ti


def kernel(pred_poles, pred_zeros, target_poles_list, target_zeros_list):
    b, np_, _ = pred_poles.shape
    ntp = target_poles_list.shape[1]
    ntz = target_zeros_list.shape[1]

    pprc, ppic = _split_rows(pred_poles)
    zprc, zpic = _split_rows(pred_zeros)
    ptrr, ptir = _split_rows(target_poles_list)
    ztrr, ztir = _split_rows(target_zeros_list)

    pred_spec = pl.BlockSpec((1, 1, np_), lambda i: (i, 0, 0))
    ptgt_spec = pl.BlockSpec((1, 1, ntp), lambda i: (i, 0, 0))
    ztgt_spec = pl.BlockSpec((1, 1, ntz), lambda i: (i, 0, 0))
    out_spec = pl.BlockSpec((1, 1, 1), lambda i: (i, 0, 0))

    pole_losses, zero_losses = pl.pallas_call(
        functools.partial(_both_kernel, cb=512, ntp=ntp, ntz=ntz),
        grid=(b,),
        in_specs=[pred_spec, pred_spec, ptgt_spec, ptgt_spec,
                  pred_spec, pred_spec, ztgt_spec, ztgt_spec],
        out_specs=[out_spec, out_spec],
        out_shape=[jax.ShapeDtypeStruct((b, 1, 1), jnp.float32),
                   jax.ShapeDtypeStruct((b, 1, 1), jnp.float32)],
        compiler_params=pltpu.CompilerParams(
            dimension_semantics=("parallel",),
        ),
    )(pprc, ppic, ptrr, ptir, zprc, zpic, ztrr, ztir)

    return jnp.mean(pole_losses) + 0.5 * jnp.mean(zero_losses)


# speedup vs baseline: 3.3272x; 1.0378x over previous
import functools

import jax
import jax.numpy as jnp
from jax.experimental import pallas as pl
from jax.experimental.pallas import tpu as pltpu


def _one_chamfer(pr_row, pi_row, tr_all, ti_all, cb, nt):
    pr = jnp.transpose(pr_row, (1, 0))
    pi = jnp.transpose(pi_row, (1, 0))
    pn = pr * pr + pi * pi
    valid = pn > 1e-12
    prm = jnp.where(valid, pr, jnp.inf)
    pim = jnp.where(valid, pi, jnp.inf)
    acc = None
    t2p_sum = jnp.float32(0.0)
    for c in range(nt // cb):
        tr = tr_all[:, c * cb:(c + 1) * cb]
        ti = ti_all[:, c * cb:(c + 1) * cb]
        dr = prm - tr
        di = pim - ti
        dist = dr * dr + di * di
        t2p_sum = t2p_sum + jnp.sum(jnp.min(dist, axis=0))
        acc = dist if acc is None else jnp.minimum(acc, dist)
    rowmin = jnp.min(acc, axis=1, keepdims=True)
    p2t_sum = jnp.sum(jnp.where(valid, rowmin, 0.0))
    cnt = jnp.sum(valid.astype(jnp.float32))
    return p2t_sum / jnp.maximum(cnt, 1.0) + t2p_sum / nt


def _both_kernel(stacked, out, total, *, cb, ntz, nb):
    i = pl.program_id(0)
    a = stacked[0]
    pole = _one_chamfer(a[0:1], a[1:2], a[2:3], a[3:4], cb, a.shape[1])
    zero = _one_chamfer(a[4:5], a[5:6], a[6:7, :ntz], a[7:8, :ntz], cb, ntz)
    step = (pole + 0.5 * zero) * (1.0 / nb)

    @pl.when(i == 0)
    def _init():
        total[0, 0] = step

    @pl.when(i != 0)
    def _acc():
        total[0, 0] = total[0, 0] + step

    @pl.when(i == nb - 1)
    def _final():
        out[...] = jnp.reshape(total[0, 0], (1, 1))


def kernel(pred_poles, pred_zeros, target_poles_list, target_zeros_list):
    b, np_, _ = pred_poles.shape
    ntz = target_zeros_list.shape[1]

    pp = jnp.transpose(pred_poles, (0, 2, 1))
    tp = jnp.transpose(target_poles_list, (0, 2, 1))
    zp = jnp.transpose(pred_zeros, (0, 2, 1))
    tz = jnp.transpose(target_zeros_list, (0, 2, 1))
    tz = jnp.pad(tz, ((0, 0), (0, 0), (0, np_ - ntz)),
                 constant_values=1e30)
    stacked = jnp.concatenate([pp, tp, zp, tz], axis=1)

    total = pl.pallas_call(
        functools.partial(_both_kernel, cb=512, ntz=ntz, nb=b),
        grid=(b,),
        in_specs=[pl.BlockSpec((1, 8, np_), lambda i: (i, 0, 0))],
        out_specs=pl.BlockSpec((1, 1), lambda i: (0, 0)),
        out_shape=jax.ShapeDtypeStruct((1, 1), jnp.float32),
        scratch_shapes=[pltpu.SMEM((1, 1), jnp.float32)],
        compiler_params=pltpu.CompilerParams(
            dimension_semantics=("arbitrary",),
        ),
    )(stacked)

    return total[0, 0]


# speedup vs baseline: 3.3392x; 1.0036x over previous
import functools

import jax
import jax.numpy as jnp
from jax.experimental import pallas as pl
from jax.experimental.pallas import tpu as pltpu


def _one_chamfer(pr_row, pi_row, tr_all, ti_all, cb, nt):
    pr = jnp.transpose(pr_row, (1, 0))
    pi = jnp.transpose(pi_row, (1, 0))
    pn = pr * pr + pi * pi
    valid = pn > 1e-12
    prm = jnp.where(valid, pr, jnp.inf)
    pim = jnp.where(valid, pi, jnp.inf)
    acc = None
    t2p_sum = jnp.float32(0.0)
    for c in range(nt // cb):
        tr = tr_all[:, c * cb:(c + 1) * cb]
        ti = ti_all[:, c * cb:(c + 1) * cb]
        dr = prm - tr
        di = pim - ti
        dist = dr * dr + di * di
        t2p_sum = t2p_sum + jnp.sum(jnp.min(dist, axis=0))
        acc = dist if acc is None else jnp.minimum(acc, dist)
    rowmin = jnp.min(acc, axis=1, keepdims=True)
    p2t_sum = jnp.sum(jnp.where(valid, rowmin, 0.0))
    cnt = jnp.sum(valid.astype(jnp.float32))
    return p2t_sum / jnp.maximum(cnt, 1.0) + t2p_sum / nt


def _both_kernel(stacked, out, total, *, cb, ntz, nb):
    i = pl.program_id(0)
    a = stacked[0]
    pole = _one_chamfer(a[0:1], a[1:2], a[2:3], a[3:4], cb, a.shape[1])
    zero = _one_chamfer(a[4:5], a[5:6], a[6:7, :ntz], a[7:8, :ntz], cb, ntz)
    step = (pole + 0.5 * zero) * (1.0 / nb)

    @pl.when(i == 0)
    def _init():
        total[0, 0] = step

    @pl.when(i != 0)
    def _acc():
        total[0, 0] = total[0, 0] + step

    @pl.when(i == nb - 1)
    def _final():
        out[...] = jnp.reshape(total[0, 0], (1, 1))


def kernel(pred_poles, pred_zeros, target_poles_list, target_zeros_list):
    b, np_, _ = pred_poles.shape
    ntz = target_zeros_list.shape[1]

    pp = jnp.transpose(pred_poles, (0, 2, 1))
    tp = jnp.transpose(target_poles_list, (0, 2, 1))
    zp = jnp.transpose(pred_zeros, (0, 2, 1))
    tz = jnp.transpose(target_zeros_list, (0, 2, 1))
    tz = jnp.pad(tz, ((0, 0), (0, 0), (0, np_ - ntz)),
                 constant_values=1e30)
    stacked = jnp.concatenate([pp, tp, zp, tz], axis=1)

    total = pl.pallas_call(
        functools.partial(_both_kernel, cb=1024, ntz=ntz, nb=b),
        grid=(b,),
        in_specs=[pl.BlockSpec((1, 8, np_), lambda i: (i, 0, 0))],
        out_specs=pl.BlockSpec((1, 1), lambda i: (0, 0)),
        out_shape=jax.ShapeDtypeStruct((1, 1), jnp.float32),
        scratch_shapes=[pltpu.SMEM((1, 1), jnp.float32)],
        compiler_params=pltpu.CompilerParams(
            dimension_semantics=("arbitrary",),
        ),
    )(stacked)

    return total[0, 0]
